# baseline (speedup 1.0000x reference)
import jax
import jax.numpy as jnp
from jax.experimental import pallas as pl
from jax.experimental.pallas import tpu as pltpu

_PRE_NMS_THRESH = 0.05
_PRE_NMS_TOP_N = 1000
_NMS_THRESH = 0.6
_POST_TOP_N = 100
_PAD = 1024


def _clip_decode(l, t, r, b, x, y, wid, hgt):
    x1 = jnp.minimum(jnp.maximum(x - l, 0.0), wid - 1.0)
    y1 = jnp.minimum(jnp.maximum(y - t, 0.0), hgt - 1.0)
    x2 = jnp.minimum(jnp.maximum(x + r, 0.0), wid - 1.0)
    y2 = jnp.minimum(jnp.maximum(y + b, 0.0), hgt - 1.0)
    return x1, y1, x2, y2


def _decode_nms_kernel(reg_ref, locx_ref, locy_ref, lab_ref, ts_ref,
                       wid_ref, hgt_ref, regt_ref, locxt_ref, locyt_ref,
                       labt_ref, widt_ref, hgtt_ref,
                       boxes_ref, keep_ref, score_ref, s_ref):
    l = reg_ref[:, 0, :]
    t = reg_ref[:, 1, :]
    r = reg_ref[:, 2, :]
    b = reg_ref[:, 3, :]
    x = locx_ref[:, 0, :]
    y = locy_ref[:, 0, :]
    wid = wid_ref[:, 0, :]
    hgt = hgt_ref[:, 0, :]
    x1, y1, x2, y2 = _clip_decode(l, t, r, b, x, y, wid, hgt)
    boxes_ref[:, 0, :] = x1
    boxes_ref[:, 1, :] = y1
    boxes_ref[:, 2, :] = x2
    boxes_ref[:, 3, :] = y2

    ts = ts_ref[:, 0, :]
    valid = (ts > 0.0) & ((x2 - x1) >= 0.0) & ((y2 - y1) >= 0.0)
    score_ref[:, 0, :] = jnp.where(valid, jnp.sqrt(jnp.maximum(ts, 1e-12)), 0.0)

    off = lab_ref[:, 0, :] * (jnp.maximum(wid, hgt) + 1.0)
    ox1 = x1 + off
    oy1 = y1 + off
    ox2 = x2 + off
    oy2 = y2 + off
    area = jnp.maximum(ox2 - ox1 + 1.0, 0.0) * jnp.maximum(oy2 - oy1 + 1.0, 0.0)

    regt = regt_ref[...][0]
    cx = locxt_ref[...][0]
    cy = locyt_ref[...][0]
    cwid = widt_ref[...][0]
    chgt = hgtt_ref[...][0]
    cx1, cy1, cx2, cy2 = _clip_decode(
        regt[:, 0:1], regt[:, 1:2], regt[:, 2:3], regt[:, 3:4],
        cx, cy, cwid, chgt)
    coff = labt_ref[...][0] * (jnp.maximum(cwid, chgt) + 1.0)
    px1 = cx1 + coff
    py1 = cy1 + coff
    px2 = cx2 + coff
    py2 = cy2 + coff
    parea = jnp.maximum(px2 - px1 + 1.0, 0.0) * jnp.maximum(py2 - py1 + 1.0, 0.0)

    ix1 = jnp.maximum(px1, ox1)
    iy1 = jnp.maximum(py1, oy1)
    ix2 = jnp.minimum(px2, ox2)
    iy2 = jnp.minimum(py2, oy2)
    inter = jnp.maximum(ix2 - ix1 + 1.0, 0.0) * jnp.maximum(iy2 - iy1 + 1.0, 0.0)
    union = parea + area - inter
    iou = inter / jnp.maximum(union, 1e-6)
    rows = jax.lax.broadcasted_iota(jnp.int32, (_PAD, _PAD), 0)
    cols = jax.lax.broadcasted_iota(jnp.int32, (_PAD, _PAD), 1)
    s_ref[...] = ((iou > _NMS_THRESH) & (cols > rows)).astype(jnp.float32)

    keep_ref[:, 0, :] = valid.astype(jnp.float32)
    lane = jax.lax.broadcasted_iota(jnp.int32, (1, _PAD), 1)

    def body(i, carry):
        row = s_ref[pl.ds(i, 1), :]
        k = keep_ref[:, 0, :]
        ki = jnp.sum(jnp.where(lane == i, k, 0.0), axis=1, keepdims=True)
        keep_ref[:, 0, :] = k * (1.0 - row * ki)
        return carry

    jax.lax.fori_loop(0, _PAD, body, 0)


def kernel(locations, box_cls, box_regression, centerness, image_sizes):
    N, C, H, W = box_cls.shape
    HW = H * W
    cls = jax.nn.sigmoid(jnp.transpose(box_cls, (0, 2, 3, 1)).reshape(N, -1, C))
    cent = jax.nn.sigmoid(jnp.transpose(centerness, (0, 2, 3, 1)).reshape(N, -1))
    candidate = cls > _PRE_NMS_THRESH
    scores = cls * cent[:, :, None]
    flat = jnp.where(candidate, scores, 0.0).reshape(N, -1)
    ts, ti = jax.lax.top_k(flat, _PRE_NMS_TOP_N)
    loc_idx = ti // C
    labels = ti % C + 1

    reg = jnp.transpose(box_regression, (0, 2, 3, 1)).reshape(N, HW, 4)
    reg_sel = jnp.take_along_axis(reg, loc_idx[..., None], axis=1)
    locx = locations[:, 0][loc_idx]
    locy = locations[:, 1][loc_idx]

    pad = _PAD - _PRE_NMS_TOP_N

    def padl(a):
        return jnp.pad(a, ((0, 0), (0, pad)))

    lab_f = labels.astype(jnp.float32)
    reg_p = jnp.pad(reg_sel.transpose(0, 2, 1), ((0, 0), (0, 0), (0, pad)))
    regt_p = jnp.pad(reg_sel, ((0, 0), (0, pad), (0, 0)))
    locx_p = padl(locx)
    locy_p = padl(locy)
    lab_p = padl(lab_f)
    ts_p = padl(ts)
    wid_b = jnp.broadcast_to(
        image_sizes[:, 1].astype(jnp.float32)[:, None], (N, _PAD))
    hgt_b = jnp.broadcast_to(
        image_sizes[:, 0].astype(jnp.float32)[:, None], (N, _PAD))

    def row3(a):
        return a[:, None, :]

    def col3(a):
        return a[:, :, None]

    spec_r4 = pl.BlockSpec((1, 4, _PAD), lambda n: (n, 0, 0))
    spec_row = pl.BlockSpec((1, 1, _PAD), lambda n: (n, 0, 0))
    spec_c4 = pl.BlockSpec((1, _PAD, 4), lambda n: (n, 0, 0))
    spec_col = pl.BlockSpec((1, _PAD, 1), lambda n: (n, 0, 0))

    boxes, keep, sc = pl.pallas_call(
        _decode_nms_kernel,
        grid=(N,),
        in_specs=[spec_r4, spec_row, spec_row, spec_row, spec_row,
                  spec_row, spec_row, spec_c4, spec_col, spec_col,
                  spec_col, spec_col, spec_col],
        out_specs=[spec_r4, spec_row, spec_row],
        out_shape=(jax.ShapeDtypeStruct((N, 4, _PAD), jnp.float32),
                   jax.ShapeDtypeStruct((N, 1, _PAD), jnp.float32),
                   jax.ShapeDtypeStruct((N, 1, _PAD), jnp.float32)),
        scratch_shapes=[pltpu.VMEM((_PAD, _PAD), jnp.float32)],
    )(reg_p, row3(locx_p), row3(locy_p), row3(lab_p), row3(ts_p),
      row3(wid_b), row3(hgt_b), regt_p, col3(locx_p), col3(locy_p),
      col3(lab_p), col3(wid_b), col3(hgt_b))

    masked = keep[:, 0, :] * sc[:, 0, :]
    fs, fidx = jax.lax.top_k(masked, _POST_TOP_N)
    fboxes = jnp.take_along_axis(
        boxes.transpose(0, 2, 1), fidx[..., None], axis=1)
    flabels = jnp.take_along_axis(padl(labels), fidx, axis=1)
    return fboxes, fs, flabels


# speedup vs baseline: 1.0002x; 1.0002x over previous
import jax
import jax.numpy as jnp
from jax.experimental import pallas as pl
from jax.experimental.pallas import tpu as pltpu

_PRE_NMS_THRESH = 0.05
_PRE_NMS_TOP_N = 1000
_NMS_THRESH = 0.6
_POST_TOP_N = 100
_PAD = 1024


def _clip_decode(l, t, r, b, x, y, wid, hgt):
    x1 = jnp.minimum(jnp.maximum(x - l, 0.0), wid - 1.0)
    y1 = jnp.minimum(jnp.maximum(y - t, 0.0), hgt - 1.0)
    x2 = jnp.minimum(jnp.maximum(x + r, 0.0), wid - 1.0)
    y2 = jnp.minimum(jnp.maximum(y + b, 0.0), hgt - 1.0)
    return x1, y1, x2, y2


def _decode_nms_kernel(reg_ref, locx_ref, locy_ref, lab_ref, ts_ref,
                       wid_ref, hgt_ref, regt_ref, locxt_ref, locyt_ref,
                       labt_ref, widt_ref, hgtt_ref,
                       boxes_ref, keep_ref, score_ref, s_ref):
    l = reg_ref[:, 0, :]
    t = reg_ref[:, 1, :]
    r = reg_ref[:, 2, :]
    b = reg_ref[:, 3, :]
    x = locx_ref[:, 0, :]
    y = locy_ref[:, 0, :]
    wid = wid_ref[:, 0, :]
    hgt = hgt_ref[:, 0, :]
    x1, y1, x2, y2 = _clip_decode(l, t, r, b, x, y, wid, hgt)
    boxes_ref[:, 0, :] = x1
    boxes_ref[:, 1, :] = y1
    boxes_ref[:, 2, :] = x2
    boxes_ref[:, 3, :] = y2

    ts = ts_ref[:, 0, :]
    valid = (ts > 0.0) & ((x2 - x1) >= 0.0) & ((y2 - y1) >= 0.0)
    score_ref[:, 0, :] = jnp.where(valid, jnp.sqrt(jnp.maximum(ts, 1e-12)), 0.0)

    off = lab_ref[:, 0, :] * (jnp.maximum(wid, hgt) + 1.0)
    ox1 = x1 + off
    oy1 = y1 + off
    ox2 = x2 + off
    oy2 = y2 + off
    area = jnp.maximum(ox2 - ox1 + 1.0, 0.0) * jnp.maximum(oy2 - oy1 + 1.0, 0.0)

    regt = regt_ref[...][0]
    cx = locxt_ref[...][0]
    cy = locyt_ref[...][0]
    cwid = widt_ref[...][0]
    chgt = hgtt_ref[...][0]
    cx1, cy1, cx2, cy2 = _clip_decode(
        regt[:, 0:1], regt[:, 1:2], regt[:, 2:3], regt[:, 3:4],
        cx, cy, cwid, chgt)
    coff = labt_ref[...][0] * (jnp.maximum(cwid, chgt) + 1.0)
    px1 = cx1 + coff
    py1 = cy1 + coff
    px2 = cx2 + coff
    py2 = cy2 + coff
    parea = jnp.maximum(px2 - px1 + 1.0, 0.0) * jnp.maximum(py2 - py1 + 1.0, 0.0)

    ix1 = jnp.maximum(px1, ox1)
    iy1 = jnp.maximum(py1, oy1)
    ix2 = jnp.minimum(px2, ox2)
    iy2 = jnp.minimum(py2, oy2)
    inter = jnp.maximum(ix2 - ix1 + 1.0, 0.0) * jnp.maximum(iy2 - iy1 + 1.0, 0.0)
    union = parea + area - inter
    iou = inter / jnp.maximum(union, 1e-6)
    rows = jax.lax.broadcasted_iota(jnp.int32, (_PAD, _PAD), 0)
    cols = jax.lax.broadcasted_iota(jnp.int32, (_PAD, _PAD), 1)
    s_ref[...] = ((iou > _NMS_THRESH) & (cols > rows)).astype(jnp.float32)

    lane = jax.lax.broadcasted_iota(jnp.int32, (1, _PAD), 1)

    def body(o, k):
        base = o * 8
        rows = s_ref[pl.ds(base, 8), :]
        for u in range(8):
            i = base + u
            row = rows[u:u + 1, :]
            ki = jnp.sum(jnp.where(lane == i, k, 0.0), axis=1, keepdims=True)
            k = k * (1.0 - row * ki)
        return k

    keep_ref[:, 0, :] = jax.lax.fori_loop(
        0, _PAD // 8, body, valid.astype(jnp.float32))


def kernel(locations, box_cls, box_regression, centerness, image_sizes):
    N, C, H, W = box_cls.shape
    HW = H * W
    cls = jax.nn.sigmoid(jnp.transpose(box_cls, (0, 2, 3, 1)).reshape(N, -1, C))
    cent = jax.nn.sigmoid(jnp.transpose(centerness, (0, 2, 3, 1)).reshape(N, -1))
    candidate = cls > _PRE_NMS_THRESH
    scores = cls * cent[:, :, None]
    flat = jnp.where(candidate, scores, 0.0).reshape(N, -1)
    ts, ti = jax.lax.top_k(flat, _PRE_NMS_TOP_N)
    loc_idx = ti // C
    labels = ti % C + 1

    reg = jnp.transpose(box_regression, (0, 2, 3, 1)).reshape(N, HW, 4)
    reg_sel = jnp.take_along_axis(reg, loc_idx[..., None], axis=1)
    locx = locations[:, 0][loc_idx]
    locy = locations[:, 1][loc_idx]

    pad = _PAD - _PRE_NMS_TOP_N

    def padl(a):
        return jnp.pad(a, ((0, 0), (0, pad)))

    lab_f = labels.astype(jnp.float32)
    reg_p = jnp.pad(reg_sel.transpose(0, 2, 1), ((0, 0), (0, 0), (0, pad)))
    regt_p = jnp.pad(reg_sel, ((0, 0), (0, pad), (0, 0)))
    locx_p = padl(locx)
    locy_p = padl(locy)
    lab_p = padl(lab_f)
    ts_p = padl(ts)
    wid_b = jnp.broadcast_to(
        image_sizes[:, 1].astype(jnp.float32)[:, None], (N, _PAD))
    hgt_b = jnp.broadcast_to(
        image_sizes[:, 0].astype(jnp.float32)[:, None], (N, _PAD))

    def row3(a):
        return a[:, None, :]

    def col3(a):
        return a[:, :, None]

    spec_r4 = pl.BlockSpec((1, 4, _PAD), lambda n: (n, 0, 0))
    spec_row = pl.BlockSpec((1, 1, _PAD), lambda n: (n, 0, 0))
    spec_c4 = pl.BlockSpec((1, _PAD, 4), lambda n: (n, 0, 0))
    spec_col = pl.BlockSpec((1, _PAD, 1), lambda n: (n, 0, 0))

    boxes, keep, sc = pl.pallas_call(
        _decode_nms_kernel,
        grid=(N,),
        in_specs=[spec_r4, spec_row, spec_row, spec_row, spec_row,
                  spec_row, spec_row, spec_c4, spec_col, spec_col,
                  spec_col, spec_col, spec_col],
        out_specs=[spec_r4, spec_row, spec_row],
        out_shape=(jax.ShapeDtypeStruct((N, 4, _PAD), jnp.float32),
                   jax.ShapeDtypeStruct((N, 1, _PAD), jnp.float32),
                   jax.ShapeDtypeStruct((N, 1, _PAD), jnp.float32)),
        scratch_shapes=[pltpu.VMEM((_PAD, _PAD), jnp.float32)],
    )(reg_p, row3(locx_p), row3(locy_p), row3(lab_p), row3(ts_p),
      row3(wid_b), row3(hgt_b), regt_p, col3(locx_p), col3(locy_p),
      col3(lab_p), col3(wid_b), col3(hgt_b))

    masked = keep[:, 0, :] * sc[:, 0, :]
    fs, fidx = jax.lax.top_k(masked, _POST_TOP_N)
    fboxes = jnp.take_along_axis(
        boxes.transpose(0, 2, 1), fidx[..., None], axis=1)
    flabels = jnp.take_along_axis(padl(labels), fidx, axis=1)
    return fboxes, fs, flabels
